# baseline (speedup 1.0000x reference)
import functools

import jax
import jax.numpy as jnp
from jax import lax
from jax.experimental import pallas as pl
from jax.experimental.pallas import tpu as pltpu
from jax.experimental.pallas import tpu_sc as plsc

_NUM_EMB = 1024
_DIM = 64
_ROWS = 9216
_ROW_BLOCK = 1024
_NW = 32
_B_PER_W = _ROWS // _NW


def _fused_body(z_ref, w_ref, idx_ref, zq_ref):
    zb = z_ref[...]
    wb = w_ref[...]
    mm2 = lax.dot_general(zb + zb, wb, (((1,), (1,)), ((), ())))
    sumz = jnp.sum(zb * zb, axis=1, keepdims=True)
    sw = jnp.sum(wb * wb, axis=1)
    chunk = 128
    m_run = (sumz - mm2[:, :chunk]) + sw[None, :chunk]
    id_run = jnp.zeros_like(m_run)
    for c in range(1, _NUM_EMB // chunk):
        d = (sumz - mm2[:, c * chunk:(c + 1) * chunk]) + sw[None, c * chunk:(c + 1) * chunk]
        upd = d < m_run
        m_run = jnp.where(upd, d, m_run)
        id_run = jnp.where(upd, float(c), id_run)
    mins = jnp.min(m_run, axis=1, keepdims=True)
    lane = lax.broadcasted_iota(jnp.int32, m_run.shape, 1).astype(jnp.float32)
    colidx = id_run * float(chunk) + lane
    cand = jnp.where(m_run == mins, colidx, float(_NUM_EMB))
    idxf = jnp.min(cand, axis=1, keepdims=True)
    idx_ref[...] = idxf[:, 0].astype(jnp.int32)
    winner = jnp.where(cand == idxf, 1.0, 0.0)
    zq = None
    for c in range(_NUM_EMB // chunk):
        mask_c = jnp.where(id_run == float(c), winner, 0.0)
        part = lax.dot_general(mask_c, wb[c * chunk:(c + 1) * chunk, :],
                               (((1,), (0,)), ((), ())),
                               precision=lax.Precision.HIGHEST)
        zq = part if zq is None else zq + part
    zq_ref[...] = zq


def _tc_fused(zf, W, interpret=False):
    grid = _ROWS // _ROW_BLOCK
    return pl.pallas_call(
        _fused_body,
        grid=(grid,),
        in_specs=[
            pl.BlockSpec((_ROW_BLOCK, _DIM), lambda i: (i, 0)),
            pl.BlockSpec((_NUM_EMB, _DIM), lambda i: (0, 0)),
        ],
        out_specs=[
            pl.BlockSpec((_ROW_BLOCK,), lambda i: (i,)),
            pl.BlockSpec((_ROW_BLOCK, _DIM), lambda i: (i, 0)),
        ],
        out_shape=[
            jax.ShapeDtypeStruct((_ROWS,), jnp.int32),
            jax.ShapeDtypeStruct((_ROWS, _DIM), jnp.float32),
        ],
        interpret=interpret,
    )(zf, W)


def _argmin_body(z_ref, w_ref, idx_ref):
    zb = z_ref[...]
    wb = w_ref[...]
    mm2 = lax.dot_general(zb + zb, wb, (((1,), (1,)), ((), ())))
    sumz = jnp.sum(zb * zb, axis=1, keepdims=True)
    sw = jnp.sum(wb * wb, axis=1)
    chunk = 128
    m_run = (sumz - mm2[:, :chunk]) + sw[None, :chunk]
    id_run = jnp.zeros_like(m_run)
    for c in range(1, _NUM_EMB // chunk):
        d = (sumz - mm2[:, c * chunk:(c + 1) * chunk]) + sw[None, c * chunk:(c + 1) * chunk]
        upd = d < m_run
        m_run = jnp.where(upd, d, m_run)
        id_run = jnp.where(upd, float(c), id_run)
    mins = jnp.min(m_run, axis=1, keepdims=True)
    lane = lax.broadcasted_iota(jnp.int32, m_run.shape, 1).astype(jnp.float32)
    colidx = id_run * float(chunk) + lane
    cand = jnp.where(m_run == mins, colidx, float(_NUM_EMB))
    idx_ref[...] = jnp.min(cand, axis=1).astype(jnp.int32)


def _tc_argmin(zf, W, interpret=False):
    grid = _ROWS // _ROW_BLOCK
    return pl.pallas_call(
        _argmin_body,
        grid=(grid,),
        in_specs=[
            pl.BlockSpec((_ROW_BLOCK, _DIM), lambda i: (i, 0)),
            pl.BlockSpec((_NUM_EMB, _DIM), lambda i: (0, 0)),
        ],
        out_specs=pl.BlockSpec((_ROW_BLOCK,), lambda i: (i,)),
        out_shape=jax.ShapeDtypeStruct((_ROWS,), jnp.int32),
        interpret=interpret,
    )(zf, W)


@functools.lru_cache(maxsize=1)
def _make_sc_gather():
    mesh = plsc.VectorSubcoreMesh(core_axis_name="c", subcore_axis_name="s")

    @functools.partial(
        pl.kernel,
        mesh=mesh,
        out_type=jax.ShapeDtypeStruct((_ROWS, _DIM), jnp.float32),
        scratch_types=[
            pltpu.VMEM((_B_PER_W,), jnp.int32),
            pltpu.VMEM((_B_PER_W, _DIM), jnp.float32),
            pltpu.SemaphoreType.DMA,
        ],
        compiler_params=pltpu.CompilerParams(use_tc_tiling_on_sc=False),
    )
    def _sc_gather(table_hbm, idx_hbm, out_hbm, idx_v, rows_v, sem):
        wid = lax.axis_index("s") * 2 + lax.axis_index("c")
        base = wid * _B_PER_W
        pltpu.sync_copy(idx_hbm.at[pl.ds(base, _B_PER_W)], idx_v)
        pltpu.async_copy(table_hbm.at[idx_v], rows_v, sem).wait()
        pltpu.sync_copy(rows_v, out_hbm.at[pl.ds(base, _B_PER_W)])

    return _sc_gather


def kernel(z, W):
    zf = z.reshape(-1, _DIM)
    idx, zq = _tc_fused(zf, W)
    return zq.reshape(z.shape), idx


# speedup vs baseline: 1.9122x; 1.9122x over previous
import functools

import jax
import jax.numpy as jnp
from jax import lax
from jax.experimental import pallas as pl
from jax.experimental.pallas import tpu as pltpu
from jax.experimental.pallas import tpu_sc as plsc

_NUM_EMB = 1024
_DIM = 64
_ROWS = 9216
_ROW_BLOCK = 3072
_NW = 32
_B_PER_W = _ROWS // _NW


def _fused_body(z_ref, w_ref, idx_ref, zq_ref):
    zb = z_ref[...]
    wb = w_ref[...]
    mm2 = lax.dot_general(zb + zb, wb, (((1,), (1,)), ((), ())))
    sumz = jnp.sum(zb * zb, axis=1, keepdims=True)
    sw = jnp.sum(wb * wb, axis=1)
    chunk = 128
    m_run = (sumz - mm2[:, :chunk]) + sw[None, :chunk]
    id_run = jnp.zeros_like(m_run)
    for c in range(1, _NUM_EMB // chunk):
        d = (sumz - mm2[:, c * chunk:(c + 1) * chunk]) + sw[None, c * chunk:(c + 1) * chunk]
        upd = d < m_run
        m_run = jnp.where(upd, d, m_run)
        id_run = jnp.where(upd, float(c), id_run)
    mins = jnp.min(m_run, axis=1, keepdims=True)
    lane = lax.broadcasted_iota(jnp.int32, m_run.shape, 1).astype(jnp.float32)
    colidx = id_run * float(chunk) + lane
    cand = jnp.where(m_run == mins, colidx, float(_NUM_EMB))
    idxf = jnp.min(cand, axis=1, keepdims=True)
    idxi = idxf.astype(jnp.int32)
    onehot = jnp.where(
        lax.broadcasted_iota(jnp.int32, (_ROW_BLOCK, _NUM_EMB), 1) == idxi,
        1.0, 0.0)
    zq_ref[...] = lax.dot_general(onehot, wb, (((1,), (0,)), ((), ())))
    col = lax.broadcasted_iota(jnp.int32, (2, _NUM_EMB), 1)
    hilo = jnp.where(lax.broadcasted_iota(jnp.int32, (2, _NUM_EMB), 0) == 0,
                     col // 32, col % 32).astype(jnp.float32)
    idxrow = lax.dot_general(hilo, onehot, (((1,), (1,)), ((), ())))
    idx_ref[...] = (idxrow[0] * 32.0 + idxrow[1]).astype(jnp.int32)


def _tc_fused(zf, W, interpret=False):
    grid = _ROWS // _ROW_BLOCK
    return pl.pallas_call(
        _fused_body,
        grid=(grid,),
        in_specs=[
            pl.BlockSpec((_ROW_BLOCK, _DIM), lambda i: (i, 0)),
            pl.BlockSpec((_NUM_EMB, _DIM), lambda i: (0, 0)),
        ],
        out_specs=[
            pl.BlockSpec((_ROW_BLOCK,), lambda i: (i,)),
            pl.BlockSpec((_ROW_BLOCK, _DIM), lambda i: (i, 0)),
        ],
        out_shape=[
            jax.ShapeDtypeStruct((_ROWS,), jnp.int32),
            jax.ShapeDtypeStruct((_ROWS, _DIM), jnp.float32),
        ],
        interpret=interpret,
    )(zf, W)


def _argmin_body(z_ref, w_ref, idx_ref):
    zb = z_ref[...]
    wb = w_ref[...]
    mm2 = lax.dot_general(zb + zb, wb, (((1,), (1,)), ((), ())))
    sumz = jnp.sum(zb * zb, axis=1, keepdims=True)
    sw = jnp.sum(wb * wb, axis=1)
    chunk = 128
    m_run = (sumz - mm2[:, :chunk]) + sw[None, :chunk]
    id_run = jnp.zeros_like(m_run)
    for c in range(1, _NUM_EMB // chunk):
        d = (sumz - mm2[:, c * chunk:(c + 1) * chunk]) + sw[None, c * chunk:(c + 1) * chunk]
        upd = d < m_run
        m_run = jnp.where(upd, d, m_run)
        id_run = jnp.where(upd, float(c), id_run)
    mins = jnp.min(m_run, axis=1, keepdims=True)
    lane = lax.broadcasted_iota(jnp.int32, m_run.shape, 1).astype(jnp.float32)
    colidx = id_run * float(chunk) + lane
    cand = jnp.where(m_run == mins, colidx, float(_NUM_EMB))
    idx_ref[...] = jnp.min(cand, axis=1).astype(jnp.int32)


def _tc_argmin(zf, W, interpret=False):
    grid = _ROWS // _ROW_BLOCK
    return pl.pallas_call(
        _argmin_body,
        grid=(grid,),
        in_specs=[
            pl.BlockSpec((_ROW_BLOCK, _DIM), lambda i: (i, 0)),
            pl.BlockSpec((_NUM_EMB, _DIM), lambda i: (0, 0)),
        ],
        out_specs=pl.BlockSpec((_ROW_BLOCK,), lambda i: (i,)),
        out_shape=jax.ShapeDtypeStruct((_ROWS,), jnp.int32),
        interpret=interpret,
    )(zf, W)


@functools.lru_cache(maxsize=1)
def _make_sc_gather():
    mesh = plsc.VectorSubcoreMesh(core_axis_name="c", subcore_axis_name="s")

    @functools.partial(
        pl.kernel,
        mesh=mesh,
        out_type=jax.ShapeDtypeStruct((_ROWS, _DIM), jnp.float32),
        scratch_types=[
            pltpu.VMEM((_B_PER_W,), jnp.int32),
            pltpu.VMEM((_B_PER_W, _DIM), jnp.float32),
            pltpu.SemaphoreType.DMA,
        ],
        compiler_params=pltpu.CompilerParams(use_tc_tiling_on_sc=False),
    )
    def _sc_gather(table_hbm, idx_hbm, out_hbm, idx_v, rows_v, sem):
        wid = lax.axis_index("s") * 2 + lax.axis_index("c")
        base = wid * _B_PER_W
        pltpu.sync_copy(idx_hbm.at[pl.ds(base, _B_PER_W)], idx_v)
        pltpu.async_copy(table_hbm.at[idx_v], rows_v, sem).wait()
        pltpu.sync_copy(rows_v, out_hbm.at[pl.ds(base, _B_PER_W)])

    return _sc_gather


def kernel(z, W):
    zf = z.reshape(-1, _DIM)
    idx, zq = _tc_fused(zf, W)
    return zq.reshape(z.shape), idx
